# baseline (speedup 1.0000x reference)
import jax
import jax.numpy as jnp
from jax.experimental import pallas as pl
from jax.experimental.pallas import tpu as pltpu

N, M, E, D = 16, 512, 768, 1024
CHUNK = 128
C = M // CHUNK


def _main_body(lens_ref, x_ref, w_ref, b_ref, q_in_ref, wq_ref, bq_ref,
               out2_ref, dots_ref, q_scratch):
    n = pl.program_id(0)
    c = pl.program_id(1)

    @pl.when(jnp.logical_and(n == 0, c == 0))
    def _():
        qz = jnp.dot(q_in_ref[...].astype(jnp.bfloat16),
                     wq_ref[...].astype(jnp.bfloat16),
                     preferred_element_type=jnp.float32) + bq_ref[...]
        q_scratch[...] = jnp.tanh(qz)

    L = lens_ref[n]
    base = c * CHUNK
    q = q_scratch[...]

    @pl.when(L <= base)
    def _():
        out2_ref[0] = jnp.zeros((CHUNK, E), jnp.float32)
        tb = jnp.tanh(b_ref[...]).astype(jnp.bfloat16).astype(jnp.float32)
        qb = q.astype(jnp.bfloat16).astype(jnp.float32)
        d_pad = jnp.sum(tb * qb)
        dots_ref[0] = jnp.full((CHUNK, 1), d_pad, jnp.float32)

    @pl.when(L > base)
    def _():
        row_ids = base + jax.lax.broadcasted_iota(jnp.int32, (CHUNK, 1), 0)
        mask = (row_ids < L).astype(jnp.float32)
        xm = x_ref[0] * mask
        out2_ref[0] = xm
        z = jnp.dot(xm.astype(jnp.bfloat16), w_ref[...].astype(jnp.bfloat16),
                    preferred_element_type=jnp.float32) + b_ref[...]
        t = jnp.tanh(z)
        d = jnp.dot(t.astype(jnp.bfloat16), q.T.astype(jnp.bfloat16),
                    preferred_element_type=jnp.float32)
        dots_ref[0] = d


def _combine_body(dots_ref, p_ref, lens_ref, out_ref):
    d = dots_ref[...][:, :, 0]
    m1 = jnp.max(d, axis=1, keepdims=True)
    e1 = jnp.exp(d - m1)
    a = e1 / jnp.sum(e1, axis=1, keepdims=True)
    logits = a * (p_ref[...] * 10.0)
    g = jnp.max(logits)
    e2 = jnp.exp(logits - g)
    w = e2 / jnp.sum(e2)
    col_ids = jax.lax.broadcasted_iota(jnp.int32, (N, M), 1)
    w = jnp.where(col_ids < lens_ref[...], w, 0.0)
    out_ref[...] = w[:, :, None]


@jax.jit
def kernel(query, groups, probabilities, group_lens, W_nodes, b_nodes,
           W_query, b_query):
    b_nodes2 = b_nodes.reshape(1, D)
    b_query2 = b_query.reshape(1, D)

    grid_spec = pltpu.PrefetchScalarGridSpec(
        num_scalar_prefetch=1,
        grid=(N, C),
        in_specs=[
            pl.BlockSpec((1, CHUNK, E), lambda n, c, lens: (n, c, 0)),
            pl.BlockSpec((E, D), lambda n, c, lens: (0, 0)),
            pl.BlockSpec((1, D), lambda n, c, lens: (0, 0)),
            pl.BlockSpec((1, D), lambda n, c, lens: (0, 0)),
            pl.BlockSpec((D, D), lambda n, c, lens: (0, 0)),
            pl.BlockSpec((1, D), lambda n, c, lens: (0, 0)),
        ],
        out_specs=[
            pl.BlockSpec((1, CHUNK, E), lambda n, c, lens: (n, c, 0)),
            pl.BlockSpec((1, CHUNK, 1), lambda n, c, lens: (n, c, 0)),
        ],
        scratch_shapes=[pltpu.VMEM((1, D), jnp.float32)],
    )
    out2, dots = pl.pallas_call(
        _main_body,
        grid_spec=grid_spec,
        out_shape=[
            jax.ShapeDtypeStruct((N, M, E), jnp.float32),
            jax.ShapeDtypeStruct((N, M, 1), jnp.float32),
        ],
    )(group_lens, groups, W_nodes, b_nodes2, query, W_query, b_query2)

    lens_col = group_lens.reshape(N, 1)
    prob_col = probabilities.reshape(N, 1)
    w = pl.pallas_call(
        _combine_body,
        in_specs=[
            pl.BlockSpec((N, M, 1), lambda: (0, 0, 0)),
            pl.BlockSpec((N, 1), lambda: (0, 0)),
            pl.BlockSpec((N, 1), lambda: (0, 0)),
        ],
        out_specs=pl.BlockSpec((N, M, 1), lambda: (0, 0, 0)),
        out_shape=jax.ShapeDtypeStruct((N, M, 1), jnp.float32),
    )(dots, prob_col, lens_col)

    return (w, out2)


# speedup vs baseline: 1.4383x; 1.4383x over previous
import jax
import jax.numpy as jnp
from jax.experimental import pallas as pl
from jax.experimental.pallas import tpu as pltpu

N, M, E, D = 16, 512, 768, 1024
CHUNK = 128
C = M // CHUNK


def _main_body(lens_ref, x_ref, w_ref, b_ref, q_in_ref, wq_ref, bq_ref,
               out2_ref, dots_ref, q_scratch):
    n = pl.program_id(0)

    @pl.when(n == 0)
    def _():
        qz = jnp.dot(q_in_ref[...].astype(jnp.bfloat16),
                     wq_ref[...].astype(jnp.bfloat16),
                     preferred_element_type=jnp.float32) + bq_ref[...]
        q_scratch[...] = jnp.tanh(qz)

    L = lens_ref[n]
    q = q_scratch[...]
    qt = q.T.astype(jnp.bfloat16)
    tb = jnp.tanh(b_ref[...]).astype(jnp.bfloat16).astype(jnp.float32)
    qb = q.astype(jnp.bfloat16).astype(jnp.float32)
    d_pad = jnp.sum(tb * qb)

    for c in range(C):
        base = c * CHUNK
        rows = pl.ds(base, CHUNK)

        @pl.when(L <= base)
        def _(rows=rows):
            out2_ref[0, rows, :] = jnp.zeros((CHUNK, E), jnp.float32)
            dots_ref[0, rows, :] = jnp.full((CHUNK, 1), d_pad, jnp.float32)

        @pl.when(L > base)
        def _(rows=rows, base=base):
            row_ids = base + jax.lax.broadcasted_iota(jnp.int32, (CHUNK, 1), 0)
            mask = (row_ids < L).astype(jnp.float32)
            xm = x_ref[0, rows, :] * mask
            out2_ref[0, rows, :] = xm
            z = jnp.dot(xm.astype(jnp.bfloat16),
                        w_ref[...].astype(jnp.bfloat16),
                        preferred_element_type=jnp.float32) + b_ref[...]
            t = jnp.tanh(z)
            d = jnp.dot(t.astype(jnp.bfloat16), qt,
                        preferred_element_type=jnp.float32)
            dots_ref[0, rows, :] = d


def _combine_body(dots_ref, p_ref, lens_ref, out_ref):
    d = dots_ref[...][:, :, 0]
    m1 = jnp.max(d, axis=1, keepdims=True)
    e1 = jnp.exp(d - m1)
    a = e1 / jnp.sum(e1, axis=1, keepdims=True)
    logits = a * (p_ref[...] * 10.0)
    g = jnp.max(logits)
    e2 = jnp.exp(logits - g)
    w = e2 / jnp.sum(e2)
    col_ids = jax.lax.broadcasted_iota(jnp.int32, (N, M), 1)
    w = jnp.where(col_ids < lens_ref[...], w, 0.0)
    out_ref[...] = w[:, :, None]


@jax.jit
def kernel(query, groups, probabilities, group_lens, W_nodes, b_nodes,
           W_query, b_query):
    b_nodes2 = b_nodes.reshape(1, D)
    b_query2 = b_query.reshape(1, D)

    grid_spec = pltpu.PrefetchScalarGridSpec(
        num_scalar_prefetch=1,
        grid=(N,),
        in_specs=[
            pl.BlockSpec((1, M, E), lambda n, lens: (n, 0, 0)),
            pl.BlockSpec((E, D), lambda n, lens: (0, 0)),
            pl.BlockSpec((1, D), lambda n, lens: (0, 0)),
            pl.BlockSpec((1, D), lambda n, lens: (0, 0)),
            pl.BlockSpec((D, D), lambda n, lens: (0, 0)),
            pl.BlockSpec((1, D), lambda n, lens: (0, 0)),
        ],
        out_specs=[
            pl.BlockSpec((1, M, E), lambda n, lens: (n, 0, 0)),
            pl.BlockSpec((1, M, 1), lambda n, lens: (n, 0, 0)),
        ],
        scratch_shapes=[pltpu.VMEM((1, D), jnp.float32)],
    )
    out2, dots = pl.pallas_call(
        _main_body,
        grid_spec=grid_spec,
        out_shape=[
            jax.ShapeDtypeStruct((N, M, E), jnp.float32),
            jax.ShapeDtypeStruct((N, M, 1), jnp.float32),
        ],
    )(group_lens, groups, W_nodes, b_nodes2, query, W_query, b_query2)

    lens_col = group_lens.reshape(N, 1)
    prob_col = probabilities.reshape(N, 1)
    w = pl.pallas_call(
        _combine_body,
        in_specs=[
            pl.BlockSpec((N, M, 1), lambda: (0, 0, 0)),
            pl.BlockSpec((N, 1), lambda: (0, 0)),
            pl.BlockSpec((N, 1), lambda: (0, 0)),
        ],
        out_specs=pl.BlockSpec((N, M, 1), lambda: (0, 0, 0)),
        out_shape=jax.ShapeDtypeStruct((N, M, 1), jnp.float32),
    )(dots, prob_col, lens_col)

    return (w, out2)


# speedup vs baseline: 2.0240x; 1.4072x over previous
import jax
import jax.numpy as jnp
from jax.experimental import pallas as pl
from jax.experimental.pallas import tpu as pltpu

N, M, E, D = 16, 512, 768, 1024


def _main_body(lens_ref, x_ref, w_ref, b_ref, q_in_ref, wq_ref, bq_ref,
               p_ref, lens_v_ref, out2_ref, out1_ref, q_scratch, dots_s):
    n = pl.program_id(0)

    @pl.when(n == 0)
    def _():
        qz = jnp.dot(q_in_ref[...].astype(jnp.bfloat16),
                     wq_ref[...].astype(jnp.bfloat16),
                     preferred_element_type=jnp.float32) + bq_ref[...]
        q_scratch[...] = jnp.tanh(qz)

    L = lens_ref[n]
    q = q_scratch[...]

    row_ids = jax.lax.broadcasted_iota(jnp.int32, (M, 1), 0)
    mask = (row_ids < L).astype(jnp.float32)
    xm = x_ref[0] * mask
    out2_ref[0] = xm
    z = jnp.dot(xm.astype(jnp.bfloat16), w_ref[...].astype(jnp.bfloat16),
                preferred_element_type=jnp.float32) + b_ref[...]
    t = jnp.tanh(z)
    d = jnp.dot(t.astype(jnp.bfloat16), q.T.astype(jnp.bfloat16),
                preferred_element_type=jnp.float32)

    lane_ids = jax.lax.broadcasted_iota(jnp.int32, (M, N), 1)
    dots_s[...] = jnp.where(lane_ids == n, jnp.broadcast_to(d, (M, N)),
                            dots_s[...])

    @pl.when(n == N - 1)
    def _():
        dd = dots_s[...]
        m1 = jnp.max(dd, axis=0, keepdims=True)
        e1 = jnp.exp(dd - m1)
        a = e1 / jnp.sum(e1, axis=0, keepdims=True)
        logits = a * (p_ref[...] * 10.0)
        g = jnp.max(logits)
        e2 = jnp.exp(logits - g)
        w = e2 / jnp.sum(e2)
        w = jnp.where(row_ids < lens_v_ref[...], w, 0.0)
        out1_ref[...] = w.T


@jax.jit
def kernel(query, groups, probabilities, group_lens, W_nodes, b_nodes,
           W_query, b_query):
    b_nodes2 = b_nodes.reshape(1, D)
    b_query2 = b_query.reshape(1, D)
    lens_row = group_lens.reshape(1, N)

    grid_spec = pltpu.PrefetchScalarGridSpec(
        num_scalar_prefetch=1,
        grid=(N,),
        in_specs=[
            pl.BlockSpec((1, M, E), lambda n, lens: (n, 0, 0)),
            pl.BlockSpec((E, D), lambda n, lens: (0, 0)),
            pl.BlockSpec((1, D), lambda n, lens: (0, 0)),
            pl.BlockSpec((1, D), lambda n, lens: (0, 0)),
            pl.BlockSpec((D, D), lambda n, lens: (0, 0)),
            pl.BlockSpec((1, D), lambda n, lens: (0, 0)),
            pl.BlockSpec((1, N), lambda n, lens: (0, 0)),
            pl.BlockSpec((1, N), lambda n, lens: (0, 0)),
        ],
        out_specs=[
            pl.BlockSpec((1, M, E), lambda n, lens: (n, 0, 0)),
            pl.BlockSpec((N, M), lambda n, lens: (0, 0)),
        ],
        scratch_shapes=[
            pltpu.VMEM((1, D), jnp.float32),
            pltpu.VMEM((M, N), jnp.float32),
        ],
    )
    out2, w = pl.pallas_call(
        _main_body,
        grid_spec=grid_spec,
        out_shape=[
            jax.ShapeDtypeStruct((N, M, E), jnp.float32),
            jax.ShapeDtypeStruct((N, M), jnp.float32),
        ],
    )(group_lens, groups, W_nodes, b_nodes2, query, W_query, b_query2,
      probabilities, lens_row)

    return (w.reshape(N, M, 1), out2)
